# Initial kernel scaffold; baseline (speedup 1.0000x reference)
import jax
import jax.numpy as jnp
from jax.experimental import pallas as pl


def kernel(data_batch, edge, W1, b1, g1, bt1, W2, b2, g2, bt2, W3, b3, g3, bt3):
    raise NotImplementedError("write your pallas kernel here")


# speedup vs baseline: 13.7943x; 13.7943x over previous
import functools

import jax
import jax.numpy as jnp
from jax import lax
from jax.experimental import pallas as pl
from jax.experimental.pallas import tpu as pltpu
from jax.experimental.pallas import tpu_sc as plsc

N = 10000
NP = 10240
E = 320000
NC = 2
NS = 16
NW = NC * NS
EB = 125
EJ = E // (NW * EB)
ROWS_PER_TILE = NP // NS
ZROWS = 64

_MESH = plsc.VectorSubcoreMesh(core_axis_name="c", subcore_axis_name="s")


def _fill_const(ref, rows, width, value):
    v = jnp.full((16,), value, jnp.float32)

    def body(j, carry):
        for k in range(width // 16):
            ref[j, pl.ds(k * 16, 16)] = v
        return carry

    lax.fori_loop(0, rows, body, 0)


def _deg_sc(dst2d):
    @functools.partial(
        pl.kernel,
        out_type=jax.ShapeDtypeStruct((NC, NP, 128), jnp.float32),
        mesh=_MESH,
        scratch_types=[
            pltpu.VMEM((EJ, EB), jnp.int32),
            pltpu.VMEM((EB, 128), jnp.float32),
            pltpu.VMEM((ZROWS, 128), jnp.float32),
            pltpu.VMEM_SHARED((NP, 128), jnp.float32),
        ],
    )
    def k(dst_hbm, out_hbm, idx_v, ones_v, zero_v, acc):
        c = lax.axis_index("c")
        s = lax.axis_index("s")
        w = c * NS + s
        _fill_const(ones_v, EB, 128, 1.0)
        _fill_const(zero_v, ZROWS, 128, 0.0)
        pltpu.sync_copy(dst_hbm.at[pl.ds(w * EJ, EJ)], idx_v)
        for z in range(ROWS_PER_TILE // ZROWS):
            pltpu.sync_copy(zero_v, acc.at[pl.ds(s * ROWS_PER_TILE + z * ZROWS, ZROWS)])
        plsc.subcore_barrier()

        def body(j, carry):
            pltpu.sync_copy(ones_v, acc.at[idx_v.at[j]], add=True)
            return carry

        lax.fori_loop(0, EJ, body, 0)
        plsc.subcore_barrier()
        for z in range(ROWS_PER_TILE // ZROWS):
            r0 = s * ROWS_PER_TILE + z * ZROWS
            pltpu.sync_copy(acc.at[pl.ds(r0, ZROWS)], out_hbm.at[c, pl.ds(r0, ZROWS)])

    return k(dst2d)


def _prop_sc(tables, src2d, dst2d):
    C = len(tables)

    @functools.partial(
        pl.kernel,
        out_type=[jax.ShapeDtypeStruct((NC, NP, 128), jnp.float32) for _ in range(C)],
        mesh=_MESH,
        scratch_types=[
            pltpu.VMEM((EJ, EB), jnp.int32),
            pltpu.VMEM((EJ, EB), jnp.int32),
            pltpu.VMEM((EB, 128), jnp.float32),
            pltpu.VMEM((ZROWS, 128), jnp.float32),
            pltpu.VMEM_SHARED((NP, 128), jnp.float32),
            pltpu.SemaphoreType.DMA,
        ],
    )
    def k(*refs):
        h_hbms = refs[:C]
        src_hbm, dst_hbm = refs[C], refs[C + 1]
        outs = refs[C + 2:C + 2 + C]
        src_v, dst_v, rows_v, zero_v, acc, sem = refs[C + 2 + C:]
        c = lax.axis_index("c")
        s = lax.axis_index("s")
        w = c * NS + s
        _fill_const(zero_v, ZROWS, 128, 0.0)
        pltpu.sync_copy(src_hbm.at[pl.ds(w * EJ, EJ)], src_v)
        pltpu.sync_copy(dst_hbm.at[pl.ds(w * EJ, EJ)], dst_v)
        for ci in range(C):
            for z in range(ROWS_PER_TILE // ZROWS):
                pltpu.sync_copy(
                    zero_v, acc.at[pl.ds(s * ROWS_PER_TILE + z * ZROWS, ZROWS)])
            plsc.subcore_barrier()

            def body(j, carry, ci=ci):
                pltpu.async_copy(h_hbms[ci].at[src_v.at[j]], rows_v, sem).wait()
                pltpu.sync_copy(rows_v, acc.at[dst_v.at[j]], add=True)
                return carry

            lax.fori_loop(0, EJ, body, 0)
            plsc.subcore_barrier()
            for z in range(ROWS_PER_TILE // ZROWS):
                r0 = s * ROWS_PER_TILE + z * ZROWS
                pltpu.sync_copy(acc.at[pl.ds(r0, ZROWS)],
                                outs[ci].at[c, pl.ds(r0, ZROWS)])

    return k(*tables, src2d, dst2d)


BN = 400
GRID = (N // BN,)


def _rowspec(*lead):
    nl = len(lead)
    return pl.BlockSpec(tuple(lead) + (BN, 128),
                        lambda i, nl=nl: (0,) * nl + (i, 0))


def _fullspec(shape):
    nd = len(shape)
    return pl.BlockSpec(shape, lambda i, nd=nd: (0,) * nd)


def _layer_norm(z, g, b):
    mu = jnp.mean(z, axis=-1, keepdims=True)
    var = jnp.mean((z - mu) ** 2, axis=-1, keepdims=True)
    return (z - mu) * lax.rsqrt(var + 1e-5) * g + b


def _prep_tc(degp, x):
    def body(deg_ref, x_ref, dinv_ref, xp_ref):
        d = deg_ref[0] + deg_ref[1] + 1.0
        dvb = lax.rsqrt(d)
        dinv_ref[...] = dvb
        xp_ref[...] = x_ref[...] * dvb

    return pl.pallas_call(
        body,
        grid=GRID,
        in_specs=[_rowspec(NC), _rowspec()],
        out_specs=[_rowspec(), _rowspec()],
        out_shape=[jax.ShapeDtypeStruct((N, 128), jnp.float32)] * 2,
    )(degp, x)


def _layer1_tc(S1, xp, dinv, W1, b1, g1, bt1):
    def body(S_ref, xp_ref, dv_ref, W_ref, b_ref, g_ref, bt_ref, *out_refs):
        u = (S_ref[0] + S_ref[1] + xp_ref[...]) * dv_ref[...]
        z = jnp.dot(u, W_ref[...], preferred_element_type=jnp.float32)
        z = jax.nn.relu(z + b_ref[...])
        y = _layer_norm(z, g_ref[...], bt_ref[...])
        for ci in range(8):
            out_refs[ci][...] = y[:, ci * 128:(ci + 1) * 128]

    return pl.pallas_call(
        body,
        grid=GRID,
        in_specs=[_rowspec(NC), _rowspec(), _rowspec(),
                  _fullspec((128, 1024)), _fullspec((1, 1024)),
                  _fullspec((1, 1024)), _fullspec((1, 1024))],
        out_specs=[_rowspec()] * 8,
        out_shape=[jax.ShapeDtypeStruct((N, 128), jnp.float32)] * 8,
    )(S1, xp, dinv, W1, b1, g1, bt1)


def _matmul_tc(x_chunks, Wr, dinv):
    CI = len(x_chunks)
    CO = Wr.shape[2] // 128

    def body(*refs):
        x_refs = refs[:CI]
        W_ref, dv_ref = refs[CI], refs[CI + 1]
        out_refs = refs[CI + 2:]
        acc = jnp.zeros((BN, Wr.shape[2]), jnp.float32)
        for ci in range(CI):
            acc = acc + jnp.dot(x_refs[ci][...], W_ref[ci],
                                preferred_element_type=jnp.float32)
        z = acc * dv_ref[:, 0:1]
        for co in range(CO):
            out_refs[co][...] = z[:, co * 128:(co + 1) * 128]

    return pl.pallas_call(
        body,
        grid=GRID,
        in_specs=[_rowspec()] * CI + [_fullspec(Wr.shape), _rowspec()],
        out_specs=[_rowspec()] * CO,
        out_shape=[jax.ShapeDtypeStruct((N, 128), jnp.float32)] * CO,
    )(*x_chunks, Wr, dinv)


def _epilogue_tc(S_parts, h_chunks, dinv, b, g, bt, chunk_major_out):
    C = len(h_chunks)
    D = C * 128

    def body(*refs):
        S_refs = refs[:C]
        h_refs = refs[C:2 * C]
        dv_ref, b_ref, g_ref, bt_ref = refs[2 * C:2 * C + 4]
        out_refs = refs[2 * C + 4:]
        dv = dv_ref[:, 0:1]
        parts = []
        for ci in range(C):
            zc = (S_refs[ci][0] + S_refs[ci][1] + h_refs[ci][...]) * dv
            parts.append(zc + b_ref[:, ci * 128:(ci + 1) * 128])
        z = jax.nn.relu(jnp.concatenate(parts, axis=1))
        y = _layer_norm(z, g_ref[...], bt_ref[...])
        if chunk_major_out:
            for ci in range(C):
                out_refs[ci][...] = y[:, ci * 128:(ci + 1) * 128]
        else:
            out_refs[0][...] = y

    if chunk_major_out:
        out_specs = [_rowspec()] * C
        out_shape = [jax.ShapeDtypeStruct((N, 128), jnp.float32)] * C
    else:
        out_specs = [pl.BlockSpec((BN, D), lambda i: (i, 0))]
        out_shape = [jax.ShapeDtypeStruct((N, D), jnp.float32)]
    res = pl.pallas_call(
        body,
        grid=GRID,
        in_specs=[_rowspec(NC)] * C + [_rowspec()] * C
                 + [_rowspec(), _fullspec((1, D)), _fullspec((1, D)),
                    _fullspec((1, D))],
        out_specs=out_specs,
        out_shape=out_shape,
    )(*S_parts, *h_chunks, dinv, b, g, bt)
    return res


def kernel(data_batch, edge, W1, b1, g1, bt1, W2, b2, g2, bt2, W3, b3, g3, bt3):
    src2d = edge[0].reshape(E // EB, EB)
    dst2d = edge[1].reshape(E // EB, EB)

    degp = _deg_sc(dst2d)
    dinv, xp = _prep_tc(degp, data_batch)

    (S1,) = _prop_sc([xp], src2d, dst2d)
    x1_chunks = _layer1_tc(S1, xp, dinv, W1, b1.reshape(1, -1),
                           g1.reshape(1, -1), bt1.reshape(1, -1))

    h2_chunks = _matmul_tc(x1_chunks, W2.reshape(8, 128, 512), dinv)
    S2_parts = _prop_sc(list(h2_chunks), src2d, dst2d)
    x2_chunks = _epilogue_tc(S2_parts, h2_chunks, dinv, b2.reshape(1, -1),
                             g2.reshape(1, -1), bt2.reshape(1, -1), True)

    h3_chunks = _matmul_tc(x2_chunks, W3.reshape(4, 128, 256), dinv)
    S3_parts = _prop_sc(list(h3_chunks), src2d, dst2d)
    (x3,) = _epilogue_tc(S3_parts, h3_chunks, dinv, b3.reshape(1, -1),
                         g3.reshape(1, -1), bt3.reshape(1, -1), False)
    return x3
